# Initial kernel scaffold; baseline (speedup 1.0000x reference)
import jax
import jax.numpy as jnp
from jax.experimental import pallas as pl


def kernel(X, word_embedding, pos_embedding):
    raise NotImplementedError("write your pallas kernel here")


# speedup vs baseline: 2.1122x; 2.1122x over previous
import functools

import jax
import jax.numpy as jnp
from jax import lax
from jax.experimental import pallas as pl
from jax.experimental.pallas import tpu as pltpu
from jax.experimental.pallas import tpu_sc as plsc

NC = 2
NS = 16
NW = NC * NS
LANES = 16


def _make_kernel(B, L, V, D):
    R = B * L
    assert R % NW == 0
    RPW = R // NW
    CH = 128
    assert RPW % CH == 0
    NCH = RPW // CH
    TROWS = V * L
    TPS = -(-TROWS // (NS * 8)) * 8
    TPAD = TPS * NS

    mesh = plsc.VectorSubcoreMesh(core_axis_name="c", subcore_axis_name="s")

    @functools.partial(
        pl.kernel,
        mesh=mesh,
        out_type=jax.ShapeDtypeStruct((R, D), jnp.float32),
        scratch_types=[
            pltpu.VMEM((V, D), jnp.float32),
            pltpu.VMEM((L, D), jnp.float32),
            pltpu.VMEM((TPS, D), jnp.float32),
            pltpu.HBM((NC, TPAD, D), jnp.float32),
            pltpu.VMEM((RPW,), jnp.int32),
            pltpu.VMEM((CH, D), jnp.float32),
            pltpu.SemaphoreType.DMA,
        ],
    )
    def k(x_hbm, word_hbm, pos_hbm, out_hbm,
          word_v, pos_v, slab_v, table_h, idx_v, stage_v, sem):
        cid = lax.axis_index("c")
        sid = lax.axis_index("s")
        wid = sid * NC + cid
        base = wid * RPW

        pltpu.sync_copy(word_hbm, word_v)
        pltpu.sync_copy(pos_hbm, pos_v)

        def build_row(i, carry):
            t = sid * TPS + i
            v = jnp.minimum(t // L, V - 1)
            p = lax.rem(t, L)
            for d in range(D // LANES):
                sl = pl.ds(d * LANES, LANES)
                slab_v[i, sl] = word_v[v, sl] + pos_v[p, sl]
            return carry

        lax.fori_loop(0, TPS, build_row, 0)
        pltpu.sync_copy(slab_v, table_h.at[cid].at[pl.ds(sid * TPS, TPS)])
        plsc.subcore_barrier()

        pltpu.sync_copy(x_hbm.at[pl.ds(base, RPW)], idx_v)
        lane = lax.iota(jnp.int32, LANES)

        def fuse(j, carry):
            off = pl.multiple_of(j * LANES, LANES)
            sl = pl.ds(off, LANES)
            idx_v[sl] = idx_v[sl] * L + lax.rem(off + lane, L)
            return carry

        lax.fori_loop(0, RPW // LANES, fuse, 0)

        def chunk(ci, carry):
            off = pl.multiple_of(ci * CH, CH)
            pltpu.async_copy(
                table_h.at[cid].at[idx_v.at[pl.ds(off, CH)]], stage_v,
                sem).wait()
            pltpu.sync_copy(stage_v, out_hbm.at[pl.ds(base + off, CH)])
            return carry

        lax.fori_loop(0, NCH, chunk, 0)

    return k


def kernel(X, word_embedding, pos_embedding):
    B, L = X.shape
    V, D = word_embedding.shape
    k = _make_kernel(B, L, V, D)
    x_flat = X.reshape(-1).astype(jnp.int32)
    out = k(x_flat, word_embedding, pos_embedding)
    return out.reshape(B, L, D)


# speedup vs baseline: 2.1179x; 1.0027x over previous
import functools

import jax
import jax.numpy as jnp
from jax import lax
from jax.experimental import pallas as pl
from jax.experimental.pallas import tpu as pltpu
from jax.experimental.pallas import tpu_sc as plsc

NC = 2
NS = 16
NW = NC * NS
LANES = 16


def _make_kernel(B, L, V, D):
    R = B * L
    assert R % NW == 0
    RPW = R // NW
    CH = 128
    assert RPW % CH == 0
    NCH = RPW // CH
    TROWS = V * L
    TPS = -(-TROWS // (NS * 8)) * 8
    TPAD = TPS * NS

    mesh = plsc.VectorSubcoreMesh(core_axis_name="c", subcore_axis_name="s")

    @functools.partial(
        pl.kernel,
        mesh=mesh,
        out_type=jax.ShapeDtypeStruct((R, D), jnp.float32),
        scratch_types=[
            pltpu.VMEM((V, D), jnp.float32),
            pltpu.VMEM((L, D), jnp.float32),
            pltpu.VMEM((TPS, D), jnp.float32),
            pltpu.HBM((NC, TPAD, D), jnp.float32),
            pltpu.VMEM((RPW,), jnp.int32),
            pltpu.VMEM((CH, D), jnp.float32),
            pltpu.VMEM((CH, D), jnp.float32),
            pltpu.SemaphoreType.DMA,
            pltpu.SemaphoreType.DMA,
            pltpu.SemaphoreType.DMA,
            pltpu.SemaphoreType.DMA,
        ],
    )
    def k(x_hbm, word_hbm, pos_hbm, out_hbm,
          word_v, pos_v, slab_v, table_h, idx_v,
          stage0, stage1, sg0, sg1, sw0, sw1):
        cid = lax.axis_index("c")
        sid = lax.axis_index("s")
        wid = sid * NC + cid
        base = wid * RPW

        pltpu.sync_copy(word_hbm, word_v)
        pltpu.sync_copy(pos_hbm, pos_v)

        def build_row(i, carry):
            t = sid * TPS + i
            v = jnp.minimum(t // L, V - 1)
            p = lax.rem(t, L)
            for d in range(D // LANES):
                sl = pl.ds(d * LANES, LANES)
                slab_v[i, sl] = word_v[v, sl] + pos_v[p, sl]
            return carry

        lax.fori_loop(0, TPS, build_row, 0)
        pltpu.sync_copy(slab_v, table_h.at[cid].at[pl.ds(sid * TPS, TPS)])
        plsc.subcore_barrier()

        pltpu.sync_copy(x_hbm.at[pl.ds(base, RPW)], idx_v)
        lane = lax.iota(jnp.int32, LANES)

        def fuse(j, carry):
            off = pl.multiple_of(j * LANES, LANES)
            sl = pl.ds(off, LANES)
            idx_v[sl] = idx_v[sl] * L + lax.rem(off + lane, L)
            return carry

        lax.fori_loop(0, RPW // LANES, fuse, 0)

        def g_copy(ci, buf, sem):
            off = pl.multiple_of(ci * CH, CH)
            return pltpu.make_async_copy(
                table_h.at[cid].at[idx_v.at[pl.ds(off, CH)]], buf, sem)

        def w_copy(ci, buf, sem):
            off = pl.multiple_of(ci * CH, CH)
            return pltpu.make_async_copy(
                buf, out_hbm.at[pl.ds(base + off, CH)], sem)

        G = NCH // 2
        g_copy(0, stage0, sg0).start()

        def pipe(g, carry):
            @pl.when(g > 0)
            def _():
                w_copy(2 * g - 1, stage1, sw1).wait()

            g_copy(2 * g + 1, stage1, sg1).start()
            g_copy(2 * g, stage0, sg0).wait()
            w_copy(2 * g, stage0, sw0).start()

            @pl.when(g < G - 1)
            def _():
                w_copy(2 * g, stage0, sw0).wait()
                g_copy(2 * g + 2, stage0, sg0).start()

            g_copy(2 * g + 1, stage1, sg1).wait()
            w_copy(2 * g + 1, stage1, sw1).start()
            return carry

        lax.fori_loop(0, G, pipe, 0)
        w_copy(NCH - 2, stage0, sw0).wait()
        w_copy(NCH - 1, stage1, sw1).wait()

    return k


def kernel(X, word_embedding, pos_embedding):
    B, L = X.shape
    V, D = word_embedding.shape
    k = _make_kernel(B, L, V, D)
    x_flat = X.reshape(-1).astype(jnp.int32)
    out = k(x_flat, word_embedding, pos_embedding)
    return out.reshape(B, L, D)
